# Initial kernel scaffold; baseline (speedup 1.0000x reference)
import jax
import jax.numpy as jnp
from jax.experimental import pallas as pl


def kernel(x, edge_index, W1, b1, W2, b2):
    raise NotImplementedError("write your pallas kernel here")


# speedup vs baseline: 9.0273x; 9.0273x over previous
import functools

import jax
import jax.numpy as jnp
from jax import lax
from jax.experimental import pallas as pl
from jax.experimental.pallas import tpu as pltpu
from jax.experimental.pallas import tpu_sc as plsc

N_NODES = 10000
N_EDGES = 320000
IN_CH = 128
OUT_CH = 128

NC, NS, LANES = 2, 16, 16
N_PAD = 10240
ROWS_PER_TILE = N_PAD // NS
CHUNK = 128
E_PAD = 327680
EPT = E_PAD // NS
EPT_DEG = E_PAD // (NC * NS)
ROW_BLK = 1024

_mesh = plsc.VectorSubcoreMesh(core_axis_name="c", subcore_axis_name="s")


@functools.partial(
    pl.kernel,
    out_type=jax.ShapeDtypeStruct((NC * N_PAD,), jnp.float32),
    mesh=_mesh,
    scratch_types=[
        pltpu.VMEM((EPT_DEG // CHUNK, CHUNK), jnp.int32),
        pltpu.VMEM((CHUNK,), jnp.float32),
        pltpu.VMEM_SHARED((N_PAD,), jnp.float32),
    ],
)
def _deg_kernel(dst2d_hbm, z1d_hbm, deg_out_hbm, dst_v, ones_v, deg_sh):
    c = lax.axis_index("c")
    s = lax.axis_index("s")
    t = c * NS + s
    pltpu.sync_copy(z1d_hbm, deg_sh.at[pl.ds(s * ROWS_PER_TILE, ROWS_PER_TILE)])
    pltpu.sync_copy(
        dst2d_hbm.at[pl.ds(t * (EPT_DEG // CHUNK), EPT_DEG // CHUNK)], dst_v
    )
    for i in range(CHUNK // LANES):
        ones_v[pl.ds(i * LANES, LANES)] = jnp.full((LANES,), 1.0, jnp.float32)
    plsc.subcore_barrier()

    def body(j, carry):
        pltpu.sync_copy(ones_v, deg_sh.at[dst_v.at[j]], add=True)
        return carry

    lax.fori_loop(0, EPT_DEG // CHUNK, body, 0)
    plsc.subcore_barrier()
    pltpu.sync_copy(
        deg_sh.at[pl.ds(s * ROWS_PER_TILE, ROWS_PER_TILE)],
        deg_out_hbm.at[pl.ds(c * N_PAD + s * ROWS_PER_TILE, ROWS_PER_TILE)],
    )


GRP = 16
N_GROUPS = EPT // (GRP * CHUNK)


def _make_seg_sum(split_edges):
    width = 128
    ept = EPT_DEG if split_edges else EPT

    @functools.partial(
        pl.kernel,
        out_type=jax.ShapeDtypeStruct((NC * N_PAD, width), jnp.float32),
        mesh=_mesh,
        scratch_types=[
            pltpu.VMEM((GRP * CHUNK,), jnp.int32),
            pltpu.VMEM((GRP, CHUNK), jnp.int32),
            pltpu.VMEM((CHUNK, width), jnp.float32),
            pltpu.VMEM((CHUNK, width), jnp.float32),
            pltpu.VMEM_SHARED((N_PAD, width), jnp.float32),
            pltpu.SemaphoreType.DMA,
        ],
    )
    def seg(hs_hbm, src2_hbm, dst2d_hbm, zw_hbm, acc_out_hbm,
            src_v, dst_v, rows0, rows1, acc_sh, gsem):
        c = lax.axis_index("c")
        s = lax.axis_index("s")
        if split_edges:
            src_base = (c * NS + s) * ept
            dst_row_base = (c * NS + s) * (ept // CHUNK)
        else:
            src_base = c * E_PAD + s * ept
            dst_row_base = s * (ept // CHUNK)
        pltpu.sync_copy(zw_hbm, acc_sh.at[pl.ds(s * ROWS_PER_TILE, ROWS_PER_TILE)])
        plsc.subcore_barrier()

        def gather_start(j, buf):
            return pltpu.async_copy(
                hs_hbm.at[src_v.at[pl.ds(j * CHUNK, CHUNK)]], buf, gsem
            )

        def gather_wait(buf):
            pltpu.make_async_copy(hs_hbm.at[src_v.at[pl.ds(0, CHUNK)]], buf,
                                  gsem).wait()

        def group(g, carry):
            pltpu.sync_copy(
                src2_hbm.at[pl.ds(src_base + g * GRP * CHUNK, GRP * CHUNK)], src_v)
            pltpu.sync_copy(dst2d_hbm.at[pl.ds(dst_row_base + g * GRP, GRP)],
                            dst_v)
            gather_start(0, rows0)

            def body(k, carry2):
                j0 = 2 * k
                gather_start(j0 + 1, rows1)
                gather_wait(rows0)
                pltpu.sync_copy(rows0, acc_sh.at[dst_v.at[j0]], add=True)

                @pl.when(k < GRP // 2 - 1)
                def _():
                    gather_start(j0 + 2, rows0)

                gather_wait(rows1)
                pltpu.sync_copy(rows1, acc_sh.at[dst_v.at[j0 + 1]], add=True)
                return carry2

            return lax.fori_loop(0, GRP // 2, body, carry)

        lax.fori_loop(0, ept // (GRP * CHUNK), group, 0)
        plsc.subcore_barrier()
        pltpu.sync_copy(
            acc_sh.at[pl.ds(s * ROWS_PER_TILE, ROWS_PER_TILE)],
            acc_out_hbm.at[pl.ds(c * N_PAD + s * ROWS_PER_TILE, ROWS_PER_TILE)],
        )

    return seg


_seg_sum_feat = _make_seg_sum(split_edges=False)
_seg_sum_part = _make_seg_sum(split_edges=True)


def _mm1_body(x_ref, w1_ref, deg_ref, hs_ref, dis_ref):
    deg = deg_ref[0] + deg_ref[1] + 1.0
    dis = lax.rsqrt(deg)
    dis_ref[...] = dis
    h = jnp.dot(x_ref[...], w1_ref[...], preferred_element_type=jnp.float32)
    hs = h * dis[:, None]
    hs_ref[0] = hs[:, :128]
    hs_ref[1] = hs[:, 128:]


def _mm2_body(acc_ref, hs_ref, dis_ref, b1_ref, w2_ref, hs2_ref):
    dis = dis_ref[...]
    b1 = b1_ref[...]
    h0 = jax.nn.relu(dis[:, None] * (acc_ref[0] + hs_ref[0]) + b1[None, :128])
    h1 = jax.nn.relu(dis[:, None] * (acc_ref[1] + hs_ref[1]) + b1[None, 128:])
    h = jnp.concatenate([h0, h1], axis=1)
    hs2 = jnp.dot(h, w2_ref[...], preferred_element_type=jnp.float32)
    hs2_ref[...] = hs2 * dis[:, None]


def _fin_body(acc_ref, hs_ref, dis_ref, b2_ref, out_ref):
    dis = dis_ref[...]
    acc = acc_ref[0] + acc_ref[1]
    out_ref[...] = dis[:, None] * (acc + hs_ref[...]) + b2_ref[...][None, :]


def _row_grid():
    return N_PAD // ROW_BLK


def _tc_mm1(x_pad, W1, deg2):
    return pl.pallas_call(
        _mm1_body,
        grid=(_row_grid(),),
        in_specs=[
            pl.BlockSpec((ROW_BLK, IN_CH), lambda i: (i, 0)),
            pl.BlockSpec((IN_CH, 256), lambda i: (0, 0)),
            pl.BlockSpec((2, ROW_BLK), lambda i: (0, i)),
        ],
        out_specs=[
            pl.BlockSpec((2, ROW_BLK, 128), lambda i: (0, i, 0)),
            pl.BlockSpec((ROW_BLK,), lambda i: (i,)),
        ],
        out_shape=[
            jax.ShapeDtypeStruct((2, N_PAD, 128), jnp.float32),
            jax.ShapeDtypeStruct((N_PAD,), jnp.float32),
        ],
    )(x_pad, W1, deg2)


def _tc_mm2(acc1, hs1, dis, b1, W2):
    return pl.pallas_call(
        _mm2_body,
        grid=(_row_grid(),),
        in_specs=[
            pl.BlockSpec((2, ROW_BLK, 128), lambda i: (0, i, 0)),
            pl.BlockSpec((2, ROW_BLK, 128), lambda i: (0, i, 0)),
            pl.BlockSpec((ROW_BLK,), lambda i: (i,)),
            pl.BlockSpec((256,), lambda i: (0,)),
            pl.BlockSpec((256, 128), lambda i: (0, 0)),
        ],
        out_specs=pl.BlockSpec((ROW_BLK, 128), lambda i: (i, 0)),
        out_shape=jax.ShapeDtypeStruct((N_PAD, 128), jnp.float32),
    )(acc1, hs1, dis, b1, W2)


def _tc_fin(acc2, hs2, dis, b2):
    return pl.pallas_call(
        _fin_body,
        grid=(_row_grid(),),
        in_specs=[
            pl.BlockSpec((2, ROW_BLK, 128), lambda i: (0, i, 0)),
            pl.BlockSpec((ROW_BLK, 128), lambda i: (i, 0)),
            pl.BlockSpec((ROW_BLK,), lambda i: (i,)),
            pl.BlockSpec((OUT_CH,), lambda i: (0,)),
        ],
        out_specs=pl.BlockSpec((ROW_BLK, OUT_CH), lambda i: (i, 0)),
        out_shape=jax.ShapeDtypeStruct((N_PAD, OUT_CH), jnp.float32),
    )(acc2, hs2, dis, b2)


def kernel(x, edge_index, W1, b1, W2, b2):
    ei = edge_index.astype(jnp.int32)
    pad_e = E_PAD - N_EDGES
    src = jnp.concatenate([ei[0], jnp.full((pad_e,), N_NODES, jnp.int32)])
    dst = jnp.concatenate([ei[1], jnp.full((pad_e,), N_NODES, jnp.int32)])
    src2 = jnp.concatenate([src, src + N_PAD])
    dst2d = dst.reshape(E_PAD // CHUNK, CHUNK)

    x_pad = jnp.pad(x, ((0, N_PAD - N_NODES), (0, 0)))
    z1d = jnp.zeros((ROWS_PER_TILE,), jnp.float32)
    z128 = jnp.zeros((ROWS_PER_TILE, 128), jnp.float32)

    deg2 = _deg_kernel(dst2d, z1d).reshape(2, N_PAD)

    hs1, dis = _tc_mm1(x_pad, W1, deg2)
    acc1 = _seg_sum_feat(hs1.reshape(NC * N_PAD, 128), src2, dst2d, z128)
    acc1 = acc1.reshape(2, N_PAD, 128)

    hs2 = _tc_mm2(acc1, hs1, dis, b1, W2)
    acc2 = _seg_sum_part(hs2, src, dst2d, z128)
    acc2 = acc2.reshape(2, N_PAD, 128)

    out = _tc_fin(acc2, hs2, dis, b2)
    return out[:N_NODES]
